# baseline (speedup 1.0000x reference)
import functools

import jax
import jax.numpy as jnp
import numpy as np
from jax import lax
from jax.experimental import pallas as pl
from jax.experimental.pallas import tpu as pltpu
from jax.experimental.pallas import tpu_sc as plsc

B, C, H, W = 16, 768, 32, 32
HW = H * W
NROWS = B * C
RANGES = (0.1, 0.3)

NC, NS = 2, 16
NWORK = NC * NS

CH = 40
ZCH = 24


def _static_plan():
    rng = np.random.RandomState(0)
    mix = []
    for i in range(0, B - 1, 2):
        frac = rng.rand() * (RANGES[1] - RANGES[0]) + RANGES[0]
        num_first = int(C * frac)
        perm = rng.permutation(C)
        mix.append((i, perm[:num_first].copy()))
    num_first = int(C * (rng.rand() * (RANGES[1] - RANGES[0]) + RANGES[0]))
    num_second = int(C * (rng.rand() * (RANGES[1] - RANGES[0]) + RANGES[0]))
    perm = rng.permutation(C)
    drop = perm[num_first:num_first + num_second].copy()

    src_b = np.tile(np.arange(B, dtype=np.int64)[:, None], (1, C))
    for i, chans in mix:
        src_b[i, chans] = i + 1
    keep = np.ones((B, C), dtype=bool)
    keep[:, drop] = False

    rows = np.arange(NROWS, dtype=np.int64).reshape(B, C)
    src_row = src_b * C + np.arange(C, dtype=np.int64)[None, :]
    gdst = rows[keep]
    gsrc = src_row[keep]
    zdst = rows[~keep]
    return gsrc.astype(np.int32), gdst.astype(np.int32), zdst.astype(np.int32)


def _pad_to(a, n):
    return np.concatenate([a, np.full(n - a.size, a[-1], a.dtype)]) if a.size < n else a


def _build_index_tables():
    gsrc, gdst, zdst = _static_plan()
    nch = max(1, -(-(-(-gsrc.size // NWORK)) // CH))
    per_w = nch * CH
    gsrc = _pad_to(gsrc, NWORK * per_w).reshape(NWORK, nch, CH)
    gdst = _pad_to(gdst, NWORK * per_w).reshape(NWORK, nch, CH)
    nzch = max(1, -(-(-(-zdst.size // NWORK)) // ZCH))
    zper_w = nzch * ZCH
    zdst = _pad_to(zdst, NWORK * zper_w).reshape(NWORK, nzch, ZCH)
    return gsrc, gdst, zdst, nch, nzch


_GSRC, _GDST, _ZDST, _NCH, _NZCH = _build_index_tables()


def _body(ytab, gsrc_h, gdst_h, zdst_h, zeros_h, out,
          gsrc_v, gdst_v, zdst_v, zbuf, buf0, buf1,
          sg0, sg1, ss0, ss1, sz):
    w = lax.axis_index("s") * NC + lax.axis_index("c")

    pltpu.sync_copy(gsrc_h.at[w], gsrc_v)
    pltpu.sync_copy(gdst_h.at[w], gdst_v)
    pltpu.sync_copy(zdst_h.at[w], zdst_v)
    pltpu.sync_copy(zeros_h, zbuf)

    zcps = [pltpu.async_copy(zbuf, out.at[zdst_v.at[j]], sz)
            for j in range(_NZCH)]

    bufs = (buf0, buf1)
    sgs = (sg0, sg1)
    sss = (ss0, ss1)

    def gather(j, b):
        return pltpu.async_copy(ytab.at[gsrc_v.at[j]], bufs[b], sgs[b])

    def scatter(j, b):
        return pltpu.async_copy(bufs[b], out.at[gdst_v.at[j]], sss[b])

    gcps = [None, None]
    scps = [None, None]
    gcps[0] = gather(0, 0)
    if _NCH > 1:
        gcps[1] = gather(1, 1)
    for j in range(_NCH):
        b = j % 2
        gcps[b].wait()
        scps[b] = scatter(j, b)
        if j + 2 < _NCH:
            scps[b].wait()
            gcps[b] = gather(j + 2, b)
    for cp in scps:
        if cp is not None:
            cp.wait()
    for cp in zcps:
        cp.wait()


def kernel(y):
    ytab = y.reshape(NROWS, 8, 128)
    mesh = plsc.VectorSubcoreMesh(core_axis_name="c", subcore_axis_name="s",
                                  num_cores=NC, num_subcores=NS)
    run = pl.kernel(
        _body,
        out_type=jax.ShapeDtypeStruct((NROWS, 8, 128), jnp.float32),
        mesh=mesh,
        scratch_types=[
            pltpu.VMEM((_NCH, CH), jnp.int32),
            pltpu.VMEM((_NCH, CH), jnp.int32),
            pltpu.VMEM((_NZCH, ZCH), jnp.int32),
            pltpu.VMEM((ZCH, 8, 128), jnp.float32),
            pltpu.VMEM((CH, 8, 128), jnp.float32),
            pltpu.VMEM((CH, 8, 128), jnp.float32),
            pltpu.SemaphoreType.DMA,
            pltpu.SemaphoreType.DMA,
            pltpu.SemaphoreType.DMA,
            pltpu.SemaphoreType.DMA,
            pltpu.SemaphoreType.DMA,
        ],
    )
    out = run(ytab,
              jnp.asarray(_GSRC), jnp.asarray(_GDST), jnp.asarray(_ZDST),
              jnp.zeros((ZCH, 8, 128), jnp.float32))
    return out.reshape(B, C, H, W)


# speedup vs baseline: 2.5540x; 2.5540x over previous
import functools

import jax
import jax.numpy as jnp
import numpy as np
from jax import lax
from jax.experimental import pallas as pl
from jax.experimental.pallas import tpu as pltpu
from jax.experimental.pallas import tpu_sc as plsc

B, C, H, W = 16, 768, 32, 32
RANGES = (0.1, 0.3)

NC, NS = 2, 16
NWORK = NC * NS
NPAIR = B // 2
QPP = NWORK // NPAIR

CT = C // 128
BPI = H * (W // 8) * CT
SPAN = BPI // QPP
CH = 24
NCHUNK = SPAN // CH
NBLK = B * BPI


def _static_plan():
    rng = np.random.RandomState(0)
    mix = []
    for i in range(0, B - 1, 2):
        frac = rng.rand() * (RANGES[1] - RANGES[0]) + RANGES[0]
        num_first = int(C * frac)
        perm = rng.permutation(C)
        mix.append(perm[:num_first].copy())
    num_first = int(C * (rng.rand() * (RANGES[1] - RANGES[0]) + RANGES[0]))
    num_second = int(C * (rng.rand() * (RANGES[1] - RANGES[0]) + RANGES[0]))
    perm = rng.permutation(C)
    drop = perm[num_first:num_first + num_second].copy()

    keep = np.ones(C, np.float32)
    keep[drop] = 0.0
    w0 = np.tile(keep, (NPAIR, 1))
    w1 = np.zeros((NPAIR, C), np.float32)
    for k, chans in enumerate(mix):
        w1[k, chans] = keep[chans]
        w0[k, chans] = 0.0
    return (w0.reshape(NPAIR, CT, 128),
            w1.reshape(NPAIR, CT, 128),
            keep.reshape(CT, 128))


_W0, _W1, _KEEP = _static_plan()


def _body(a, w0_h, w1_h, kp_h, out, w0_v, w1_v, kp_v,
          be0, bo0, be1, bo1, si0, si1, so0, so1):
    wid = lax.axis_index("s") * NC + lax.axis_index("c")
    k = wid // QPP
    q = wid % QPP
    ebase = (2 * k) * BPI + q * SPAN
    obase = ebase + BPI

    pltpu.sync_copy(w0_h.at[k], w0_v)
    pltpu.sync_copy(w1_h.at[k], w1_v)
    pltpu.sync_copy(kp_h, kp_v)

    bes = (be0, be1)
    bos = (bo0, bo1)
    sis = (si0, si1)
    sos = (so0, so1)

    def start_in(j, p):
        off = j * CH
        pltpu.async_copy(a.at[pl.ds(ebase + off, CH)], bes[p], sis[p])
        pltpu.async_copy(a.at[pl.ds(obase + off, CH)], bos[p], sis[p])

    def wait_in(p):
        pltpu.make_async_copy(a.at[pl.ds(ebase, CH)], bes[p], sis[p]).wait()
        pltpu.make_async_copy(a.at[pl.ds(obase, CH)], bos[p], sis[p]).wait()

    def start_out(j, p):
        off = j * CH
        pltpu.async_copy(bes[p], out.at[pl.ds(ebase + off, CH)], sos[p])
        pltpu.async_copy(bos[p], out.at[pl.ds(obase + off, CH)], sos[p])

    def wait_out(p):
        pltpu.make_async_copy(bes[p], out.at[pl.ds(ebase, CH)], sos[p]).wait()
        pltpu.make_async_copy(bos[p], out.at[pl.ds(obase, CH)], sos[p]).wait()

    def compute(p):
        be, bo = bes[p], bos[p]
        for l in range(8):
            sl = pl.ds(l * 16, 16)

            def blk_body(m, _, sl=sl, be=be, bo=bo):
                ct = lax.rem(m, CT)
                w0v = w0_v[ct, sl]
                w1v = w1_v[ct, sl]
                kv = kp_v[ct, sl]
                for s in range(8):
                    ve = be[m, s, sl]
                    vo = bo[m, s, sl]
                    be[m, s, sl] = ve * w0v + vo * w1v
                    bo[m, s, sl] = vo * kv
                return _

            lax.fori_loop(0, CH, blk_body, None)

    start_in(0, 0)
    start_in(1, 1)

    def chunk_pair(jj, _):
        for p in (0, 1):
            j = 2 * jj + p
            wait_in(p)
            compute(p)
            start_out(j, p)
            wait_out(p)

            @pl.when(j + 2 < NCHUNK)
            def _start_next(j=j, p=p):
                start_in(j + 2, p)
        return _

    lax.fori_loop(0, NCHUNK // 2, chunk_pair, None)


def kernel(y):
    a6 = (y.transpose(0, 2, 3, 1)
           .reshape(B, H, W // 8, 8, C // 128, 128)
           .transpose(0, 1, 2, 4, 3, 5))
    a = a6.reshape(NBLK, 8, 128)
    mesh = plsc.VectorSubcoreMesh(core_axis_name="c", subcore_axis_name="s",
                                  num_cores=NC, num_subcores=NS)
    run = pl.kernel(
        _body,
        out_type=jax.ShapeDtypeStruct((NBLK, 8, 128), jnp.float32),
        mesh=mesh,
        scratch_types=[
            pltpu.VMEM((CT, 128), jnp.float32),
            pltpu.VMEM((CT, 128), jnp.float32),
            pltpu.VMEM((CT, 128), jnp.float32),
            pltpu.VMEM((CH, 8, 128), jnp.float32),
            pltpu.VMEM((CH, 8, 128), jnp.float32),
            pltpu.VMEM((CH, 8, 128), jnp.float32),
            pltpu.VMEM((CH, 8, 128), jnp.float32),
            pltpu.SemaphoreType.DMA,
            pltpu.SemaphoreType.DMA,
            pltpu.SemaphoreType.DMA,
            pltpu.SemaphoreType.DMA,
        ],
    )
    o = run(a, jnp.asarray(_W0), jnp.asarray(_W1), jnp.asarray(_KEEP))
    o6 = o.reshape(B, H, W // 8, C // 128, 8, 128)
    return (o6.transpose(0, 1, 2, 4, 3, 5)
              .reshape(B, H, W, C)
              .transpose(0, 3, 1, 2))


# speedup vs baseline: 2.7752x; 1.0866x over previous
import functools

import jax
import jax.numpy as jnp
import numpy as np
from jax import lax
from jax.experimental import pallas as pl
from jax.experimental.pallas import tpu as pltpu
from jax.experimental.pallas import tpu_sc as plsc

B, C, H, W = 16, 768, 32, 32
RANGES = (0.1, 0.3)

NC, NS = 2, 16
NWORK = NC * NS
NPAIR = B // 2
QPP = NWORK // NPAIR

CT = C // 128
BPI = H * (W // 8) * CT
SPAN = BPI // QPP
CH = 12
NCHUNK = SPAN // CH
NSLOT = 4
NBLK = B * BPI


def _static_plan():
    rng = np.random.RandomState(0)
    mix = []
    for i in range(0, B - 1, 2):
        frac = rng.rand() * (RANGES[1] - RANGES[0]) + RANGES[0]
        num_first = int(C * frac)
        perm = rng.permutation(C)
        mix.append(perm[:num_first].copy())
    num_first = int(C * (rng.rand() * (RANGES[1] - RANGES[0]) + RANGES[0]))
    num_second = int(C * (rng.rand() * (RANGES[1] - RANGES[0]) + RANGES[0]))
    perm = rng.permutation(C)
    drop = perm[num_first:num_first + num_second].copy()

    keep = np.ones(C, np.float32)
    keep[drop] = 0.0
    w0 = np.tile(keep, (NPAIR, 1))
    w1 = np.zeros((NPAIR, C), np.float32)
    for k, chans in enumerate(mix):
        w1[k, chans] = keep[chans]
        w0[k, chans] = 0.0
    return (w0.reshape(NPAIR, CT, 128),
            w1.reshape(NPAIR, CT, 128),
            keep.reshape(CT, 128))


_W0, _W1, _KEEP = _static_plan()


def _body(a, w0_h, w1_h, kp_h, out, w0_v, w1_v, kp_v,
          be0, bo0, be1, bo1, be2, bo2, be3, bo3,
          si0, si1, si2, si3, so0, so1, so2, so3):
    wid = lax.axis_index("s") * NC + lax.axis_index("c")
    k = wid // QPP
    q = wid % QPP
    ebase = (2 * k) * BPI + q * SPAN
    obase = ebase + BPI

    pltpu.sync_copy(w0_h.at[k], w0_v)
    pltpu.sync_copy(w1_h.at[k], w1_v)
    pltpu.sync_copy(kp_h, kp_v)

    bes = (be0, be1, be2, be3)
    bos = (bo0, bo1, bo2, bo3)
    sis = (si0, si1, si2, si3)
    sos = (so0, so1, so2, so3)

    def start_in(j, p):
        off = j * CH
        pltpu.async_copy(a.at[pl.ds(ebase + off, CH)], bes[p], sis[p])
        pltpu.async_copy(a.at[pl.ds(obase + off, CH)], bos[p], sis[p])

    def wait_in(p):
        pltpu.make_async_copy(a.at[pl.ds(ebase, CH)], bes[p], sis[p]).wait()
        pltpu.make_async_copy(a.at[pl.ds(obase, CH)], bos[p], sis[p]).wait()

    def start_out(j, p):
        off = j * CH
        pltpu.async_copy(bes[p], out.at[pl.ds(ebase + off, CH)], sos[p])
        pltpu.async_copy(bos[p], out.at[pl.ds(obase + off, CH)], sos[p])

    def wait_out(p):
        pltpu.make_async_copy(bes[p], out.at[pl.ds(ebase, CH)], sos[p]).wait()
        pltpu.make_async_copy(bos[p], out.at[pl.ds(obase, CH)], sos[p]).wait()

    def compute(p):
        be, bo = bes[p], bos[p]
        for l in range(8):
            sl = pl.ds(l * 16, 16)

            def blk_body(m, _, sl=sl, be=be, bo=bo):
                ct = lax.rem(m, CT)
                w0v = w0_v[ct, sl]
                w1v = w1_v[ct, sl]
                kv = kp_v[ct, sl]
                for s in range(8):
                    ve = be[m, s, sl]
                    vo = bo[m, s, sl]
                    be[m, s, sl] = ve * w0v + vo * w1v
                    bo[m, s, sl] = vo * kv
                return _

            lax.fori_loop(0, CH, blk_body, None)

    start_in(0, 0)
    start_in(1, 1)
    start_in(2, 2)

    def ring(jj, _):
        for s in range(NSLOT):
            t = jj * NSLOT + s
            wait_in(s)
            compute(s)
            start_out(t, s)
            r = (s - 1) % NSLOT
            if s == 0:
                @pl.when(jj == 0)
                def _first_fill():
                    start_in(NSLOT - 1, NSLOT - 1)

                @pl.when(jj > 0)
                def _refill0(t=t):
                    wait_out(r)
                    start_in(t + NSLOT - 1, r)
            else:
                @pl.when(t + NSLOT - 1 < NCHUNK)
                def _refill(t=t, s=s, r=r):
                    wait_out(r)
                    start_in(t + NSLOT - 1, r)
        return _

    lax.fori_loop(0, NCHUNK // NSLOT, ring, None)
    for s in range(NSLOT):
        wait_out(s)


def kernel(y):
    a6 = (y.transpose(0, 2, 3, 1)
           .reshape(B, H, W // 8, 8, C // 128, 128)
           .transpose(0, 1, 2, 4, 3, 5))
    a = a6.reshape(NBLK, 8, 128)
    mesh = plsc.VectorSubcoreMesh(core_axis_name="c", subcore_axis_name="s",
                                  num_cores=NC, num_subcores=NS)
    run = pl.kernel(
        _body,
        out_type=jax.ShapeDtypeStruct((NBLK, 8, 128), jnp.float32),
        mesh=mesh,
        scratch_types=(
            [pltpu.VMEM((CT, 128), jnp.float32)] * 3
            + [pltpu.VMEM((CH, 8, 128), jnp.float32)] * (2 * NSLOT)
            + [pltpu.SemaphoreType.DMA] * (2 * NSLOT)
        ),
    )
    o = run(a, jnp.asarray(_W0), jnp.asarray(_W1), jnp.asarray(_KEEP))
    o6 = o.reshape(B, H, W // 8, C // 128, 8, 128)
    return (o6.transpose(0, 1, 2, 4, 3, 5)
              .reshape(B, H, W, C)
              .transpose(0, 3, 1, 2))


# speedup vs baseline: 3.9779x; 1.4334x over previous
import functools

import jax
import jax.numpy as jnp
import numpy as np
from jax import lax
from jax.experimental import pallas as pl
from jax.experimental.pallas import tpu as pltpu
from jax.experimental.pallas import tpu_sc as plsc

B, C, H, W = 16, 768, 32, 32
RANGES = (0.1, 0.3)

NC, NS = 2, 16
NWORK = NC * NS
NPAIR = B // 2
QPP = NWORK // NPAIR

CT = C // 128
BPI = H * (W // 8) * CT
SPAN = BPI // QPP
CH = 12
NCHUNK = SPAN // CH
NSLOT = 4
NBLK = B * BPI


def _static_plan():
    rng = np.random.RandomState(0)
    mix = []
    for i in range(0, B - 1, 2):
        frac = rng.rand() * (RANGES[1] - RANGES[0]) + RANGES[0]
        num_first = int(C * frac)
        perm = rng.permutation(C)
        mix.append(perm[:num_first].copy())
    num_first = int(C * (rng.rand() * (RANGES[1] - RANGES[0]) + RANGES[0]))
    num_second = int(C * (rng.rand() * (RANGES[1] - RANGES[0]) + RANGES[0]))
    perm = rng.permutation(C)
    drop = perm[num_first:num_first + num_second].copy()

    keep = np.ones(C, np.float32)
    keep[drop] = 0.0
    w0 = np.tile(keep, (NPAIR, 1))
    w1 = np.zeros((NPAIR, C), np.float32)
    for k, chans in enumerate(mix):
        w1[k, chans] = keep[chans]
        w0[k, chans] = 0.0
    return (w0.reshape(NPAIR, CT, 128),
            w1.reshape(NPAIR, CT, 128),
            keep.reshape(CT, 128))


_W0, _W1, _KEEP = _static_plan()


def _body(a, w0_h, w1_h, kp_h, out, w0_v, w1_v, kp_v,
          be0, bo0, be1, bo1, be2, bo2, be3, bo3,
          si0, si1, si2, si3, so0, so1, so2, so3):
    wid = lax.axis_index("s") * NC + lax.axis_index("c")
    k = wid // QPP
    q = wid % QPP
    ebase = (2 * k) * BPI + q * SPAN
    obase = ebase + BPI

    pltpu.sync_copy(w0_h.at[k], w0_v)
    pltpu.sync_copy(w1_h.at[k], w1_v)
    pltpu.sync_copy(kp_h, kp_v)

    bes = (be0, be1, be2, be3)
    bos = (bo0, bo1, bo2, bo3)
    sis = (si0, si1, si2, si3)
    sos = (so0, so1, so2, so3)

    def start_in(j, p):
        off = j * CH
        pltpu.async_copy(a.at[pl.ds(ebase + off, CH)], bes[p], sis[p])
        pltpu.async_copy(a.at[pl.ds(obase + off, CH)], bos[p], sis[p])

    def wait_in(p):
        pltpu.make_async_copy(a.at[pl.ds(ebase, CH)], bes[p], sis[p]).wait()
        pltpu.make_async_copy(a.at[pl.ds(obase, CH)], bos[p], sis[p]).wait()

    def start_out(j, p):
        off = j * CH
        pltpu.async_copy(bes[p], out.at[pl.ds(ebase + off, CH)], sos[p])
        pltpu.async_copy(bos[p], out.at[pl.ds(obase + off, CH)], sos[p])

    def wait_out(p):
        pltpu.make_async_copy(bes[p], out.at[pl.ds(ebase, CH)], sos[p]).wait()
        pltpu.make_async_copy(bos[p], out.at[pl.ds(obase, CH)], sos[p]).wait()

    def compute(p):
        be, bo = bes[p], bos[p]
        for l in range(8):
            sl = pl.ds(l * 16, 16)

            @plsc.parallel_loop(0, CH, 1)
            def blk_body(m, sl=sl, be=be, bo=bo):
                ct = lax.rem(m, CT)
                w0v = w0_v[ct, sl]
                w1v = w1_v[ct, sl]
                kv = kp_v[ct, sl]
                for s in range(8):
                    ve = be[m, s, sl]
                    vo = bo[m, s, sl]
                    be[m, s, sl] = ve * w0v + vo * w1v
                    bo[m, s, sl] = vo * kv

    start_in(0, 0)
    start_in(1, 1)
    start_in(2, 2)

    def ring(jj, _):
        for s in range(NSLOT):
            t = jj * NSLOT + s
            wait_in(s)
            compute(s)
            start_out(t, s)
            r = (s - 1) % NSLOT
            if s == 0:
                @pl.when(jj == 0)
                def _first_fill():
                    start_in(NSLOT - 1, NSLOT - 1)

                @pl.when(jj > 0)
                def _refill0(t=t):
                    wait_out(r)
                    start_in(t + NSLOT - 1, r)
            else:
                @pl.when(t + NSLOT - 1 < NCHUNK)
                def _refill(t=t, s=s, r=r):
                    wait_out(r)
                    start_in(t + NSLOT - 1, r)
        return _

    lax.fori_loop(0, NCHUNK // NSLOT, ring, None)
    for s in range(NSLOT):
        wait_out(s)


def kernel(y):
    a6 = (y.transpose(0, 2, 3, 1)
           .reshape(B, H, W // 8, 8, C // 128, 128)
           .transpose(0, 1, 2, 4, 3, 5))
    a = a6.reshape(NBLK, 8, 128)
    mesh = plsc.VectorSubcoreMesh(core_axis_name="c", subcore_axis_name="s",
                                  num_cores=NC, num_subcores=NS)
    run = pl.kernel(
        _body,
        out_type=jax.ShapeDtypeStruct((NBLK, 8, 128), jnp.float32),
        mesh=mesh,
        scratch_types=(
            [pltpu.VMEM((CT, 128), jnp.float32)] * 3
            + [pltpu.VMEM((CH, 8, 128), jnp.float32)] * (2 * NSLOT)
            + [pltpu.SemaphoreType.DMA] * (2 * NSLOT)
        ),
    )
    o = run(a, jnp.asarray(_W0), jnp.asarray(_W1), jnp.asarray(_KEEP))
    o6 = o.reshape(B, H, W // 8, C // 128, 8, 128)
    return (o6.transpose(0, 1, 2, 4, 3, 5)
              .reshape(B, H, W, C)
              .transpose(0, 3, 1, 2))


# speedup vs baseline: 5.4627x; 1.3733x over previous
import functools

import jax
import jax.numpy as jnp
import numpy as np
from jax import lax
from jax.experimental import pallas as pl
from jax.experimental.pallas import tpu as pltpu
from jax.experimental.pallas import tpu_sc as plsc

B, C, H, W = 16, 768, 32, 32
RANGES = (0.1, 0.3)

NC, NS = 2, 16
NWORK = NC * NS
NPAIR = B // 2
QPP = NWORK // NPAIR

CT = C // 128
BPI = H * (W // 8) * CT
SPAN = BPI // QPP
CH = 12
NCHUNK = SPAN // CH
NSLOT = 4
NBLK = B * BPI


def _static_plan():
    rng = np.random.RandomState(0)
    mix = []
    for i in range(0, B - 1, 2):
        frac = rng.rand() * (RANGES[1] - RANGES[0]) + RANGES[0]
        num_first = int(C * frac)
        perm = rng.permutation(C)
        mix.append(perm[:num_first].copy())
    num_first = int(C * (rng.rand() * (RANGES[1] - RANGES[0]) + RANGES[0]))
    num_second = int(C * (rng.rand() * (RANGES[1] - RANGES[0]) + RANGES[0]))
    perm = rng.permutation(C)
    drop = perm[num_first:num_first + num_second].copy()

    keep = np.ones(C, np.float32)
    keep[drop] = 0.0
    w0 = np.tile(keep, (NPAIR, 1))
    w1 = np.zeros((NPAIR, C), np.float32)
    for k, chans in enumerate(mix):
        w1[k, chans] = keep[chans]
        w0[k, chans] = 0.0
    return (w0.reshape(NPAIR, CT, 128),
            w1.reshape(NPAIR, CT, 128),
            keep.reshape(CT, 128))


_W0, _W1, _KEEP = _static_plan()


def _body(a, w0_h, w1_h, kp_h, out, w0_v, w1_v, kp_v,
          be0, bo0, be1, bo1, be2, bo2, be3, bo3,
          si0, si1, si2, si3, so0, so1, so2, so3):
    wid = lax.axis_index("s") * NC + lax.axis_index("c")
    k = wid // QPP
    q = wid % QPP
    ebase = (2 * k) * BPI + q * SPAN
    obase = ebase + BPI

    pltpu.sync_copy(w0_h.at[k], w0_v)
    pltpu.sync_copy(w1_h.at[k], w1_v)
    pltpu.sync_copy(kp_h, kp_v)

    bes = (be0, be1, be2, be3)
    bos = (bo0, bo1, bo2, bo3)
    sis = (si0, si1, si2, si3)
    sos = (so0, so1, so2, so3)

    def start_in(j, p):
        off = j * CH
        pltpu.async_copy(a.at[pl.ds(ebase + off, CH)], bes[p], sis[p])
        pltpu.async_copy(a.at[pl.ds(obase + off, CH)], bos[p], sis[p])

    def wait_in(p):
        pltpu.make_async_copy(a.at[pl.ds(ebase, CH)], bes[p], sis[p]).wait()
        pltpu.make_async_copy(a.at[pl.ds(obase, CH)], bos[p], sis[p]).wait()

    def start_out(j, p):
        off = j * CH
        pltpu.async_copy(bes[p], out.at[pl.ds(ebase + off, CH)], sos[p])
        pltpu.async_copy(bos[p], out.at[pl.ds(obase + off, CH)], sos[p])

    def wait_out(p):
        pltpu.make_async_copy(bes[p], out.at[pl.ds(ebase, CH)], sos[p]).wait()
        pltpu.make_async_copy(bos[p], out.at[pl.ds(obase, CH)], sos[p]).wait()

    def compute(p):
        be, bo = bes[p], bos[p]

        @plsc.parallel_loop(0, CH * 8, 1, unroll=4)
        def blk_body(i, be=be, bo=bo):
            m = lax.div(i, 8)
            l = lax.rem(i, 8)
            ct = lax.rem(m, CT)
            sl = pl.ds(l * 16, 16)
            w0v = w0_v[ct, sl]
            w1v = w1_v[ct, sl]
            kv = kp_v[ct, sl]
            for s in range(8):
                ve = be[m, s, sl]
                vo = bo[m, s, sl]
                be[m, s, sl] = ve * w0v + vo * w1v
                bo[m, s, sl] = vo * kv

    start_in(0, 0)
    start_in(1, 1)
    start_in(2, 2)

    def ring(jj, _):
        for s in range(NSLOT):
            t = jj * NSLOT + s
            wait_in(s)
            compute(s)
            start_out(t, s)
            r = (s - 1) % NSLOT
            if s == 0:
                @pl.when(jj == 0)
                def _first_fill():
                    start_in(NSLOT - 1, NSLOT - 1)

                @pl.when(jj > 0)
                def _refill0(t=t):
                    wait_out(r)
                    start_in(t + NSLOT - 1, r)
            else:
                @pl.when(t + NSLOT - 1 < NCHUNK)
                def _refill(t=t, s=s, r=r):
                    wait_out(r)
                    start_in(t + NSLOT - 1, r)
        return _

    lax.fori_loop(0, NCHUNK // NSLOT, ring, None)
    for s in range(NSLOT):
        wait_out(s)


def kernel(y):
    a6 = (y.transpose(0, 2, 3, 1)
           .reshape(B, H, W // 8, 8, C // 128, 128)
           .transpose(0, 1, 2, 4, 3, 5))
    a = a6.reshape(NBLK, 8, 128)
    mesh = plsc.VectorSubcoreMesh(core_axis_name="c", subcore_axis_name="s",
                                  num_cores=NC, num_subcores=NS)
    run = pl.kernel(
        _body,
        out_type=jax.ShapeDtypeStruct((NBLK, 8, 128), jnp.float32),
        mesh=mesh,
        scratch_types=(
            [pltpu.VMEM((CT, 128), jnp.float32)] * 3
            + [pltpu.VMEM((CH, 8, 128), jnp.float32)] * (2 * NSLOT)
            + [pltpu.SemaphoreType.DMA] * (2 * NSLOT)
        ),
    )
    o = run(a, jnp.asarray(_W0), jnp.asarray(_W1), jnp.asarray(_KEEP))
    o6 = o.reshape(B, H, W // 8, C // 128, 8, 128)
    return (o6.transpose(0, 1, 2, 4, 3, 5)
              .reshape(B, H, W, C)
              .transpose(0, 3, 1, 2))
